# baseline (speedup 1.0000x reference)
import functools

import jax
import jax.numpy as jnp
from jax import lax
from jax.experimental import pallas as pl
from jax.experimental.pallas import tpu as pltpu
from jax.experimental.pallas import tpu_sc as plsc

B, S, D, V = 4096, 200, 64, 1000000
NC, NS = 2, 16
NW = NC * NS
NBLK = B // 128
NCHT = S * NBLK
NCH = NCHT // NW
NBUF = 4


def _body(xi_ref, tok_ref, pos_ref, out_ref, idxb, bufs, obufs, posv,
          isem, gsem, osem):
    wid = lax.axis_index("s") * NC + lax.axis_index("c")
    c0 = wid * NCH

    pltpu.sync_copy(pos_ref, posv)

    def sj(k):
        c = c0 + k
        s = lax.div(c, NBLK)
        return s, c - s * NBLK

    def start_idx(k):
        slot = lax.rem(k, NBUF)
        s, j = sj(k)
        pltpu.async_copy(xi_ref.at[s, pl.ds(j * 128, 128)], idxb.at[slot],
                         isem.at[slot])

    def wait_idx(slot):
        pltpu.make_async_copy(xi_ref.at[0, pl.ds(0, 128)], idxb.at[slot],
                              isem.at[slot]).wait()

    def start_gather(k):
        slot = lax.rem(k, NBUF)
        pltpu.async_copy(tok_ref.at[idxb.at[slot]], bufs.at[slot],
                         gsem.at[slot])

    def wait_gather(slot):
        pltpu.make_async_copy(tok_ref.at[idxb.at[0]], bufs.at[slot],
                              gsem.at[slot]).wait()

    def start_out(k, slot):
        s, j = sj(k)
        pltpu.async_copy(obufs.at[slot], out_ref.at[s, :, j], osem.at[slot])

    def wait_out(slot):
        pltpu.make_async_copy(obufs.at[slot], out_ref.at[0, :, 0],
                              osem.at[slot]).wait()

    start_idx(0)
    start_idx(1)
    wait_idx(0)
    start_gather(0)

    iota16 = jax.lax.iota(jnp.int32, 16)
    tiq = [lax.shift_right_logical(iota16 + 16 * q, 3) for q in range(4)]
    rrq = [lax.bitwise_and(iota16 + 16 * q, 7) for q in range(4)]

    def chunk(k, carry):
        slot = lax.rem(k, NBUF)
        slot16c = jnp.full((16,), slot, jnp.int32)

        @pl.when(k + 1 < NCH)
        def _():
            wait_idx(lax.rem(k + 1, NBUF))

            @pl.when(k >= NBUF - 1)
            def _():
                wait_out(lax.rem(k + 1, NBUF))
            start_gather(k + 1)

        @pl.when(k + 2 < NCH)
        def _():
            start_idx(k + 2)

        wait_gather(slot)

        s, _ = sj(k)
        pvq = [posv[s, pl.ds(q * 16, 16)] for q in range(4)]

        @plsc.parallel_loop(0, 128, step=1, unroll=4)
        def _t(r):
            r16 = jnp.full((16,), r, jnp.int32)
            for q in range(4):
                val = bufs[slot, r, pl.ds(q * 16, 16)] + pvq[q]
                plsc.store_scatter(obufs, [slot16c, tiq[q], rrq[q], r16], val)

        start_out(k, slot)
        return carry

    lax.fori_loop(0, NCH, chunk, 0)
    for t in range(NBUF):
        wait_out(t)


_sc_call = functools.partial(
    pl.kernel,
    out_type=jax.ShapeDtypeStruct((S, 8, NBLK, 8, 128), jnp.float32),
    mesh=plsc.VectorSubcoreMesh(
        core_axis_name="c", subcore_axis_name="s",
        num_cores=NC, num_subcores=NS),
    scratch_types=[
        pltpu.VMEM((NBUF, 128), jnp.int32),
        pltpu.VMEM((NBUF, 128, D), jnp.float32),
        pltpu.VMEM((NBUF, 8, 8, 128), jnp.float32),
        pltpu.VMEM((S, D), jnp.float32),
        pltpu.SemaphoreType.DMA((NBUF,)),
        pltpu.SemaphoreType.DMA((NBUF,)),
        pltpu.SemaphoreType.DMA((NBUF,)),
    ],
    compiler_params=pltpu.CompilerParams(
        use_tc_tiling_on_sc=False, needs_layout_passes=False),
)(_body)


def kernel(x, token_table, position_table):
    xi = jnp.transpose(x).astype(jnp.int32)
    out = _sc_call(xi, token_table, position_table)
    return jnp.transpose(out, (2, 4, 0, 1, 3)).reshape(B, S, D)


# speedup vs baseline: 1.0558x; 1.0558x over previous
import functools

import jax
import jax.numpy as jnp
from jax import lax
from jax.experimental import pallas as pl
from jax.experimental.pallas import tpu as pltpu
from jax.experimental.pallas import tpu_sc as plsc

B, S, D, V = 4096, 200, 64, 1000000
NC, NS = 2, 16
NW = NC * NS
ROWS = B * S
RPW = ROWS // NW
CH = 128
NCH = RPW // CH
NBUF = 6
PD = 3
POSREP = 320


def _body(x_ref, tok_ref, pos_ref, out_ref, idx_all, pos2, bufs, gsem, osem):
    wid = lax.axis_index("s") * NC + lax.axis_index("c")
    irow0 = wid * NCH
    orow0 = wid * RPW

    pltpu.sync_copy(x_ref.at[pl.ds(irow0, NCH)], idx_all)
    pltpu.sync_copy(pos_ref, pos2.at[pl.ds(0, S)])
    pltpu.sync_copy(pos_ref.at[pl.ds(0, POSREP - S)], pos2.at[pl.ds(S, POSREP - S)])

    def start_gather(g):
        slot = lax.rem(g, NBUF)
        pltpu.async_copy(tok_ref.at[idx_all.at[g]], bufs.at[slot], gsem.at[slot])

    def wait_gather(slot):
        pltpu.make_async_copy(
            tok_ref.at[idx_all.at[0]], bufs.at[slot], gsem.at[slot]).wait()

    def start_out(g, slot):
        pltpu.async_copy(
            bufs.at[slot], out_ref.at[pl.ds(orow0 + g * CH, CH)], osem.at[slot])

    def wait_out(slot):
        pltpu.make_async_copy(
            bufs.at[slot], out_ref.at[pl.ds(orow0, CH)], osem.at[slot]).wait()

    for g in range(PD):
        start_gather(g)

    def chunk(lg, carry):
        slot = lax.rem(lg, NBUF)
        gn = lg + PD

        @pl.when(gn < NCH)
        def _():
            @pl.when(lg >= PD)
            def _():
                wait_out(lax.rem(gn, NBUF))
            start_gather(gn)

        wait_gather(slot)

        o = lax.rem(lg * CH, S)

        @plsc.parallel_loop(0, CH, step=1, unroll=8)
        def _add(r):
            for q in range(D // 16):
                bufs[slot, r, pl.ds(q * 16, 16)] = (
                    bufs[slot, r, pl.ds(q * 16, 16)]
                    + pos2[o + r, pl.ds(q * 16, 16)])

        start_out(lg, slot)
        return carry

    lax.fori_loop(0, NCH, chunk, 0)
    for k in range(NBUF):
        wait_out(k)


_sc_call = functools.partial(
    pl.kernel,
    out_type=jax.ShapeDtypeStruct((ROWS, D), jnp.float32),
    mesh=plsc.VectorSubcoreMesh(
        core_axis_name="c", subcore_axis_name="s",
        num_cores=NC, num_subcores=NS),
    scratch_types=[
        pltpu.VMEM((NCH, CH), jnp.int32),
        pltpu.VMEM((POSREP, D), jnp.float32),
        pltpu.VMEM((NBUF, CH, D), jnp.float32),
        pltpu.SemaphoreType.DMA((NBUF,)),
        pltpu.SemaphoreType.DMA((NBUF,)),
    ],
    compiler_params=pltpu.CompilerParams(use_tc_tiling_on_sc=False),
)(_body)


def kernel(x, token_table, position_table):
    x2 = x.astype(jnp.int32).reshape(ROWS // CH, CH)
    out = _sc_call(x2, token_table, position_table)
    return out.reshape(B, S, D)
